# baseline (speedup 1.0000x reference)
import math

import jax
import jax.numpy as jnp
from jax import lax
from jax.experimental import pallas as pl
from jax.experimental.pallas import tpu as pltpu
from jax.experimental.pallas import tpu_sc as plsc

VOCAB = 1000000
D_MODEL = 64
BATCH = 4096
HIST = 200
SCALE = math.sqrt(D_MODEL)

NC = 2
NS = 16
NW = NC * NS
LANES = 16

B_TOTAL = BATCH * HIST
ROWS_PER_W = B_TOTAL // NW
CHUNK = 128
NCHUNK = ROWS_PER_W // CHUNK


def _body(x_hbm, table_hbm, out_hbm, idx_v, rows_v, sem):
    wid = lax.axis_index("s") * NC + lax.axis_index("c")
    pltpu.sync_copy(x_hbm.at[wid], idx_v)

    def chunk_step(g, carry):
        pltpu.async_copy(table_hbm.at[idx_v.at[g]], rows_v, sem).wait()

        def row_step(r, c2):
            for c4 in range(D_MODEL // LANES):
                sl = pl.ds(c4 * LANES, LANES)
                rows_v[r, sl] = rows_v[r, sl] * SCALE
            return c2

        lax.fori_loop(0, CHUNK, row_step, 0, unroll=2)

        base = (wid * ROWS_PER_W) + g * CHUNK
        pltpu.sync_copy(rows_v, out_hbm.at[pl.ds(base, CHUNK)])
        return carry

    lax.fori_loop(0, NCHUNK, chunk_step, 0)


@jax.jit
def _embed(x_grouped, weight):
    mesh = plsc.VectorSubcoreMesh(core_axis_name="c", subcore_axis_name="s")
    k = pl.kernel(
        _body,
        out_type=jax.ShapeDtypeStruct((B_TOTAL, D_MODEL), jnp.float32),
        mesh=mesh,
        scratch_types=[
            pltpu.VMEM((NCHUNK, CHUNK), jnp.int32),
            pltpu.VMEM((CHUNK, D_MODEL), jnp.float32),
            pltpu.SemaphoreType.DMA,
        ],
        compiler_params=pltpu.CompilerParams(use_tc_tiling_on_sc=False),
    )
    return k(x_grouped, weight)


def kernel(x, weight):
    x_grouped = x.reshape(NW, NCHUNK, CHUNK)
    out = _embed(x_grouped, weight)
    return out.reshape(BATCH, HIST, D_MODEL)


# speedup vs baseline: 1.1663x; 1.1663x over previous
import math

import jax
import jax.numpy as jnp
from jax import lax
from jax.experimental import pallas as pl
from jax.experimental.pallas import tpu as pltpu
from jax.experimental.pallas import tpu_sc as plsc

VOCAB = 1000000
D_MODEL = 64
BATCH = 4096
HIST = 200
SCALE = math.sqrt(D_MODEL)

NC = 2
NS = 16
NW = NC * NS
LANES = 16

B_TOTAL = BATCH * HIST
ROWS_PER_W = B_TOTAL // NW
CHUNK = 128
NCHUNK = ROWS_PER_W // CHUNK
M = 8
L = 4
NOUTER = NCHUNK // M


def _body(x_hbm, table_hbm, out_hbm, idx_v, bufs, gsem, ssem):
    wid = lax.axis_index("s") * NC + lax.axis_index("c")
    pltpu.sync_copy(x_hbm.at[wid], idx_v)
    base = wid * ROWS_PER_W

    def gather(g, slot):
        return pltpu.make_async_copy(
            table_hbm.at[idx_v.at[g]], bufs.at[slot], gsem.at[slot]
        )

    def scatter(g, slot):
        return pltpu.make_async_copy(
            bufs.at[slot], out_hbm.at[pl.ds(base + g * CHUNK, CHUNK)], ssem.at[slot]
        )

    def scale(slot):
        def row_step(r, c2):
            for c4 in range(D_MODEL // LANES):
                sl = pl.ds(c4 * LANES, LANES)
                bufs[slot, r, sl] = bufs[slot, r, sl] * SCALE
            return c2

        lax.fori_loop(0, CHUNK, row_step, 0, unroll=2)

    for s in range(L):
        gather(s, s).start()

    def turn(o, b, issue, wait_old):
        g = o * M + b
        gather(g, b).wait()
        if issue:
            si = (b + L) % M
            gn = g + L
            if wait_old:
                scatter(gn - M, si).wait()
            gather(gn, si).start()
        scale(b)
        scatter(g, b).start()

    for b in range(M):
        turn(0, b, issue=True, wait_old=(b >= M - L))

    def outer(o, carry):
        for b in range(M):
            turn(o, b, issue=True, wait_old=True)
        return carry

    lax.fori_loop(1, NOUTER - 1, outer, 0)

    for b in range(M):
        turn(NOUTER - 1, b, issue=(b < M - L), wait_old=(b < M - L))

    for b in range(M):
        scatter((NOUTER - 1) * M + b, b).wait()


@jax.jit
def _embed(x_grouped, weight):
    mesh = plsc.VectorSubcoreMesh(core_axis_name="c", subcore_axis_name="s")
    k = pl.kernel(
        _body,
        out_type=jax.ShapeDtypeStruct((B_TOTAL, D_MODEL), jnp.float32),
        mesh=mesh,
        scratch_types=[
            pltpu.VMEM((NCHUNK, CHUNK), jnp.int32),
            pltpu.VMEM((M, CHUNK, D_MODEL), jnp.float32),
            pltpu.SemaphoreType.DMA((M,)),
            pltpu.SemaphoreType.DMA((M,)),
        ],
        compiler_params=pltpu.CompilerParams(use_tc_tiling_on_sc=False),
    )
    return k(x_grouped, weight)


def kernel(x, weight):
    x_grouped = x.reshape(NW, NCHUNK, CHUNK)
    out = _embed(x_grouped, weight)
    return out.reshape(BATCH, HIST, D_MODEL)


# speedup vs baseline: 1.2004x; 1.0292x over previous
import math

import jax
import jax.numpy as jnp
from jax import lax
from jax.experimental import pallas as pl
from jax.experimental.pallas import tpu as pltpu
from jax.experimental.pallas import tpu_sc as plsc

VOCAB = 1000000
D_MODEL = 64
BATCH = 4096
HIST = 200
SCALE = math.sqrt(D_MODEL)

NC = 2
NS = 16
NW = NC * NS
LANES = 16
NVEC = D_MODEL // LANES

B_TOTAL = BATCH * HIST
ROWS_PER_W = B_TOTAL // NW
CHUNK = 128
NCHUNK = ROWS_PER_W // CHUNK

SLAB = 160
NSLAB = VOCAB // SLAB
FULL_K = 195
EXTRA_W = NSLAB - NW * FULL_K

M = 4
L = 2
NOUTER = NCHUNK // M


def _prep_body(table_hbm, t_out, inb, outb, rsem, wsem):
    wid = lax.axis_index("s") * NC + lax.axis_index("c")

    def slab_of(k):
        return wid + k * NW

    def read(k, s):
        rows = pl.ds(slab_of(k) * SLAB, SLAB)
        return pltpu.make_async_copy(table_hbm.at[rows], inb.at[s], rsem.at[s])

    def write(k, s):
        rows = pl.ds(slab_of(k) * SLAB, SLAB)
        return pltpu.make_async_copy(outb.at[s], t_out.at[rows], wsem.at[s])

    def scale_copy(s):
        def row_step(r, c2):
            for c in range(NVEC):
                sl = pl.ds(c * LANES, LANES)
                outb[s, r, sl] = inb[s, r, sl] * SCALE
            return c2

        lax.fori_loop(0, SLAB, row_step, 0, unroll=2)

    for s in range(3):
        read(s, s).start()

    def turn(k, s, issue_guard, wait_old=True):
        read(k, s).wait()
        if wait_old:
            write(k - 3, s).wait()
        scale_copy(s)
        kn = k + 3
        if issue_guard is None:
            read(kn, s).start()
        elif issue_guard:
            @pl.when(wid < EXTRA_W)
            def _():
                read(kn, s).start()
        write(k, s).start()

    for b in range(3):
        turn(b, b, None, wait_old=False)

    def outer(o, carry):
        for b in range(3):
            turn(o * 3 + b, b, None)
        return carry

    lax.fori_loop(1, 64, outer, 0)

    turn(192, 0, True)
    turn(193, 1, False)
    turn(194, 2, False)

    @pl.when(wid < EXTRA_W)
    def _():
        turn(195, 0, False)
        write(195, 0).wait()

    @pl.when(wid >= EXTRA_W)
    def _():
        write(192, 0).wait()

    write(193, 1).wait()
    write(194, 2).wait()


def _lookup_body(x_hbm, t_hbm, out_hbm, idx_v, gbuf, obuf, gsem, ssem):
    wid = lax.axis_index("s") * NC + lax.axis_index("c")
    pltpu.sync_copy(x_hbm.at[wid], idx_v)
    base = wid * ROWS_PER_W

    def gather(g, slot):
        return pltpu.make_async_copy(
            t_hbm.at[idx_v.at[g]], gbuf.at[slot], gsem.at[slot]
        )

    def scatter(g, slot):
        return pltpu.make_async_copy(
            obuf.at[slot], out_hbm.at[pl.ds(base + g * CHUNK, CHUNK)], ssem.at[slot]
        )

    def compact(gs, os_):
        def row_step(r, c2):
            for c in range(NVEC):
                sl = pl.ds(c * LANES, LANES)
                obuf[os_, r, sl] = gbuf[gs, r, sl]
            return c2

        lax.fori_loop(0, CHUNK, row_step, 0, unroll=2)

    for s in range(L):
        gather(s, s).start()

    def turn(g, b, issue, wait_old):
        gather(g, b).wait()
        if issue:
            gather(g + L, (b + L) % M).start()
        if wait_old:
            scatter(g - 2, b % 2).wait()
        compact(b, b % 2)
        scatter(g, b % 2).start()

    for b in range(M):
        turn(b, b, issue=True, wait_old=(b >= 2))

    def outer(o, carry):
        for b in range(M):
            turn(o * M + b, b, issue=True, wait_old=True)
        return carry

    lax.fori_loop(1, NOUTER - 1, outer, 0)

    for b in range(M):
        g = (NOUTER - 1) * M + b
        turn(g, b, issue=(b < M - L), wait_old=True)

    scatter(NCHUNK - 2, 0).wait()
    scatter(NCHUNK - 1, 1).wait()


@jax.jit
def _embed(x_grouped, weight):
    mesh = plsc.VectorSubcoreMesh(core_axis_name="c", subcore_axis_name="s")
    prep = pl.kernel(
        _prep_body,
        out_type=jax.ShapeDtypeStruct((VOCAB, 2 * D_MODEL), jnp.float32),
        mesh=mesh,
        scratch_types=[
            pltpu.VMEM((3, SLAB, D_MODEL), jnp.float32),
            pltpu.VMEM((3, SLAB, 2 * D_MODEL), jnp.float32),
            pltpu.SemaphoreType.DMA((3,)),
            pltpu.SemaphoreType.DMA((3,)),
        ],
        compiler_params=pltpu.CompilerParams(use_tc_tiling_on_sc=True),
    )
    t = prep(weight)
    look = pl.kernel(
        _lookup_body,
        out_type=jax.ShapeDtypeStruct((B_TOTAL, D_MODEL), jnp.float32),
        mesh=mesh,
        scratch_types=[
            pltpu.VMEM((NCHUNK, CHUNK), jnp.int32),
            pltpu.VMEM((M, CHUNK, 2 * D_MODEL), jnp.float32),
            pltpu.VMEM((2, CHUNK, D_MODEL), jnp.float32),
            pltpu.SemaphoreType.DMA((M,)),
            pltpu.SemaphoreType.DMA((2,)),
        ],
        compiler_params=pltpu.CompilerParams(use_tc_tiling_on_sc=True),
    )
    out = look(x_grouped, t)
    return out.reshape(BATCH, HIST, D_MODEL)


def kernel(x, weight):
    x_grouped = x.reshape(NW, NCHUNK, CHUNK)
    return _embed(x_grouped, weight)
